# Initial kernel scaffold; baseline (speedup 1.0000x reference)
import jax
import jax.numpy as jnp
from jax.experimental import pallas as pl


def kernel(query, key, value, proj_dir):
    raise NotImplementedError("write your pallas kernel here")


# speedup vs baseline: 3.4642x; 3.4642x over previous
import functools
import math

import jax
import jax.numpy as jnp
from jax import lax
from jax.experimental import pallas as pl
from jax.experimental.pallas import tpu as pltpu
from jax.experimental.pallas import tpu_sc as plsc

NPROJ = 8
BLK = 128
SAMP = 128
NBINS = 2 ** NPROJ


def _hash_rank_body(q_ref, k_ref, projp_ref, rq_ref, sq_ref, sk_ref,
                    bins_scr, prior_scr, fwd_scr):
    S = q_ref.shape[1]
    C = S // BLK
    off = pl.program_id(0) * S

    lane = lax.broadcasted_iota(jnp.int32, (1, BLK), 1)
    pow2 = jnp.where(lane < NPROJ, lax.shift_left(1, jnp.minimum(lane, NPROJ - 1)), 0)
    maskrow = pow2.astype(jnp.float32)
    ltri = (lax.broadcasted_iota(jnp.int32, (BLK, BLK), 0)
            < lax.broadcasted_iota(jnp.int32, (BLK, BLK), 1)).astype(jnp.bfloat16)
    bins_iota = lax.broadcasted_iota(jnp.int32, (NBINS, BLK), 0)
    mstart = (lax.broadcasted_iota(jnp.int32, (NBINS, NBINS), 1)
              < lax.broadcasted_iota(jnp.int32, (NBINS, NBINS), 0)).astype(jnp.float32)
    projp = projp_ref[...]

    iota_c = lax.broadcasted_iota(jnp.int32, (S // BLK, 1), 0)
    iota_lcol = lax.broadcasted_iota(jnp.int32, (BLK, 1), 0)

    def one(x_ref, rank_ref, sidx_ref):
        def loop_a(c, acc):
            xc = x_ref[0, pl.ds(c * BLK, BLK), :]
            proj = jnp.dot(xc, projp, preferred_element_type=jnp.float32)
            bits = (proj > 0).astype(jnp.float32)
            bint = lax.dot_general(maskrow, bits, (((1,), (1,)), ((), ())),
                                   preferred_element_type=jnp.float32)
            b = bint.astype(jnp.int32)
            h = jnp.bitwise_xor(b, lax.shift_right_logical(b, 1))
            bins_scr[pl.ds(c, 1), :] = h
            oh = (bins_iota == h).astype(jnp.float32)
            inner = lax.dot_general(oh.astype(jnp.bfloat16), ltri,
                                    (((1,), (0,)), ((), ())),
                                    preferred_element_type=jnp.float32)
            within = jnp.sum(oh * inner, axis=0, keepdims=True)
            prev = jnp.sum(oh * acc, axis=0, keepdims=True)
            prior_scr[pl.ds(c, 1), :] = within + prev
            return acc + jnp.sum(oh, axis=1, keepdims=True)

        hist = lax.fori_loop(0, C, loop_a, jnp.zeros((NBINS, 1), jnp.float32))
        hist_i = hist.astype(jnp.int32)
        hist_hi = lax.shift_right_logical(hist_i, 8).astype(jnp.float32)
        hist_lo = jnp.bitwise_and(hist_i, 255).astype(jnp.float32)
        bstart = (256.0 * lax.dot_general(mstart, hist_hi, (((1,), (0,)), ((), ())),
                                          preferred_element_type=jnp.float32)
                  + lax.dot_general(mstart, hist_lo, (((1,), (0,)), ((), ())),
                                    preferred_element_type=jnp.float32))

        fwd_scr[:, :] = jnp.zeros((C, BLK), jnp.float32)

        def loop_b(c, _):
            h = bins_scr[pl.ds(c, 1), :]
            oh = (bins_iota == h).astype(jnp.float32)
            base = jnp.sum(oh * bstart, axis=0, keepdims=True)
            local = (base + prior_scr[pl.ds(c, 1), :]).astype(jnp.int32)
            if rank_ref is not None:
                rank_ref[0, :, pl.ds(c * BLK, BLK)] = local + off
            hi = lax.shift_right_logical(local, 7)
            lo = jnp.bitwise_and(local, BLK - 1)
            oh_hi = (iota_c == hi).astype(jnp.float32)
            oh_lot = (iota_lcol == lo).astype(jnp.float32)
            jvec = lax.broadcasted_iota(jnp.int32, (1, BLK), 1).astype(jnp.float32)
            ind = lax.dot_general(oh_hi, oh_lot, (((1,), (1,)), ((), ())),
                                  preferred_element_type=jnp.float32)
            jpart = lax.dot_general(oh_hi * jvec, oh_lot, (((1,), (1,)), ((), ())),
                                    preferred_element_type=jnp.float32)
            fwd_scr[:, :] = fwd_scr[:, :] + jpart + (c * BLK).astype(jnp.float32) * ind
            return 0

        lax.fori_loop(0, C, loop_b, 0)
        sidx_ref[0, :, :] = fwd_scr[:, :].astype(jnp.int32) + off

    one(q_ref, rq_ref, sq_ref)
    one(k_ref, None, sk_ref)


def _rank_call(q3, k3, projp):
    BH, S, D = q3.shape
    C = S // BLK
    return pl.pallas_call(
        _hash_rank_body,
        grid=(BH,),
        in_specs=[
            pl.BlockSpec((1, S, D), lambda b: (b, 0, 0)),
            pl.BlockSpec((1, S, D), lambda b: (b, 0, 0)),
            pl.BlockSpec((D, BLK), lambda b: (0, 0)),
        ],
        out_specs=[
            pl.BlockSpec((1, 1, S), lambda b: (b, 0, 0)),
            pl.BlockSpec((1, C, BLK), lambda b: (b, 0, 0)),
            pl.BlockSpec((1, C, BLK), lambda b: (b, 0, 0)),
        ],
        out_shape=[jax.ShapeDtypeStruct((BH, 1, S), jnp.int32),
                   jax.ShapeDtypeStruct((BH, C, BLK), jnp.int32),
                   jax.ShapeDtypeStruct((BH, C, BLK), jnp.int32)],
        scratch_shapes=[pltpu.VMEM((C, BLK), jnp.int32),
                        pltpu.VMEM((C, BLK), jnp.float32),
                        pltpu.VMEM((C, BLK), jnp.float32)],
    )(q3, k3, projp)


def _attn_body(qs_ref, ks_ref, vs_ref, ksub_ref, vsub_ref, sdiv_ref, out_ref,
               *, scale, logw, d):
    i = pl.program_id(1)
    q = qs_ref[0][:, :d]
    k = ks_ref[0][:, :d]
    v = vs_ref[0][:, :d]
    s1 = lax.dot_general(q, k, (((1,), (1,)), ((), ())),
                         preferred_element_type=jnp.float32) * scale
    m1 = jnp.max(s1, axis=1, keepdims=True)
    p1 = jnp.exp(s1 - m1)
    l1 = jnp.sum(p1, axis=1, keepdims=True)
    a1 = jnp.dot(p1, v, preferred_element_type=jnp.float32)
    lse1 = m1 + jnp.log(l1)

    ksub = ksub_ref[0][:, :d]
    vsub = vsub_ref[0][:, :d]
    s2 = lax.dot_general(q, ksub, (((1,), (1,)), ((), ())),
                         preferred_element_type=jnp.float32) * scale
    coll = sdiv_ref[0] == i
    s2 = s2 + jnp.where(coll, jnp.finfo(jnp.float32).min, 0.0)
    m2 = jnp.max(s2, axis=1, keepdims=True)
    p2 = jnp.exp(s2 - m2)
    l2 = jnp.sum(p2, axis=1, keepdims=True)
    a2 = jnp.dot(p2, vsub, preferred_element_type=jnp.float32)
    lse2 = m2 + jnp.log(l2) + logw

    dlt = lse2 - lse1
    c1 = 1.0 / (1.0 + jnp.exp(dlt))
    c2 = 1.0 / (1.0 + jnp.exp(-dlt))
    res = c1 * (a1 / l1) + c2 * (a2 / l2)
    out_ref[0] = jnp.concatenate(
        [res, jnp.zeros((BLK, BLK - d), jnp.float32)], axis=1)


def _attn_call(qs, ks, vs, ksub, vsub, sdiv, d):
    BH, S, _ = qs.shape
    C = S // BLK
    body = functools.partial(_attn_body, scale=d ** (-0.5),
                             logw=math.log(S / SAMP), d=d)
    return pl.pallas_call(
        body,
        grid=(BH, C),
        in_specs=[
            pl.BlockSpec((1, BLK, BLK), lambda b, i: (b, i, 0)),
            pl.BlockSpec((1, BLK, BLK), lambda b, i: (b, i, 0)),
            pl.BlockSpec((1, BLK, BLK), lambda b, i: (b, i, 0)),
            pl.BlockSpec((1, SAMP, BLK), lambda b, i: (b, 0, 0)),
            pl.BlockSpec((1, SAMP, BLK), lambda b, i: (b, 0, 0)),
            pl.BlockSpec((1, 1, SAMP), lambda b, i: (b, 0, 0)),
        ],
        out_specs=pl.BlockSpec((1, BLK, BLK), lambda b, i: (b, i, 0)),
        out_shape=jax.ShapeDtypeStruct((BH, S, BLK), jnp.float32),
    )(qs, ks, vs, ksub, vsub, sdiv)


def _pad_body(x_ref, o_ref):
    x = x_ref[0]
    o_ref[0] = jnp.concatenate(
        [x, jnp.zeros((x.shape[0], BLK - x.shape[1]), jnp.float32)], axis=1)


def _pad_call(q3, k3, v3):
    BH, S, D = q3.shape
    spec = pl.BlockSpec((1, 512, D), lambda b, i: (b, i, 0))
    ospec = pl.BlockSpec((1, 512, BLK), lambda b, i: (b, i, 0))
    f = pl.pallas_call(
        _pad_body,
        grid=(BH, S // 512),
        in_specs=[spec],
        out_specs=ospec,
        out_shape=jax.ShapeDtypeStruct((BH, S, BLK), jnp.float32),
    )
    return f(q3), f(k3), f(v3)


def _sc_sort_call(qf, kf, vf, sq, sk, samp_flat):
    BHS, D = qf.shape
    BH = sq.shape[0]
    S = BHS // BH
    C = S // BLK
    mesh = plsc.VectorSubcoreMesh(core_axis_name="c", subcore_axis_name="s")

    @functools.partial(
        pl.kernel, mesh=mesh,
        out_type=[jax.ShapeDtypeStruct((BHS, D), jnp.float32)] * 3
        + [jax.ShapeDtypeStruct((BH * SAMP, D), jnp.float32)] * 2,
        scratch_types=[
            pltpu.VMEM((C, BLK), jnp.int32),
            pltpu.VMEM((BLK, D), jnp.float32),
            pltpu.VMEM((SAMP,), jnp.int32),
            pltpu.SemaphoreType.DMA,
        ],
    )
    def k2(q_hbm, k_hbm, v_hbm, sq_hbm, sk_hbm, samp_hbm,
           qs_out, ks_out, vs_out, ksub_out, vsub_out,
           perm_v, buf, sidx_v, sem):
        wid = lax.axis_index("s") * 2 + lax.axis_index("c")

        def do_array(src_hbm, dst_hbm):
            def chunk(c, _):
                pltpu.async_copy(src_hbm.at[perm_v.at[c]], buf, sem).wait()
                pltpu.sync_copy(buf, dst_hbm.at[pl.ds(wid * S + c * BLK, BLK)])
                return 0
            lax.fori_loop(0, C, chunk, 0)

        pltpu.sync_copy(sq_hbm.at[wid], perm_v)
        do_array(q_hbm, qs_out)
        pltpu.sync_copy(sk_hbm.at[wid], perm_v)
        do_array(k_hbm, ks_out)
        do_array(v_hbm, vs_out)

        pltpu.sync_copy(samp_hbm.at[wid], sidx_v)
        pltpu.async_copy(ks_out.at[sidx_v], buf, sem).wait()
        pltpu.sync_copy(buf, ksub_out.at[pl.ds(wid * SAMP, SAMP)])
        pltpu.async_copy(vs_out.at[sidx_v], buf, sem).wait()
        pltpu.sync_copy(buf, vsub_out.at[pl.ds(wid * SAMP, SAMP)])

    return k2(qf, kf, vf, sq, sk, samp_flat)


def _sc_unsort_call(attns, rq):
    BHS, D = attns.shape
    BH = rq.shape[0]
    S = BHS // BH
    C = S // BLK
    mesh = plsc.VectorSubcoreMesh(core_axis_name="c", subcore_axis_name="s")

    @functools.partial(
        pl.kernel, mesh=mesh,
        out_type=jax.ShapeDtypeStruct((BHS, D), jnp.float32),
        scratch_types=[
            pltpu.VMEM((S,), jnp.int32),
            pltpu.VMEM((BLK, D), jnp.float32),
            pltpu.SemaphoreType.DMA,
        ],
    )
    def k4(attns_hbm, rq_hbm, out_hbm, idx_v, buf, sem):
        wid = lax.axis_index("s") * 2 + lax.axis_index("c")
        pltpu.sync_copy(rq_hbm.at[wid, 0], idx_v)

        def chunk(c, _):
            pltpu.async_copy(
                attns_hbm.at[idx_v.at[pl.ds(c * BLK, BLK)]], buf, sem).wait()
            pltpu.sync_copy(buf, out_hbm.at[pl.ds(wid * S + c * BLK, BLK)])
            return 0

        lax.fori_loop(0, C, chunk, 0)

    return k4(attns, rq)


def kernel(query, key, value, proj_dir):
    B, H, S, D = query.shape
    BH = B * H
    q3 = query.reshape(BH, S, D)
    k3 = key.reshape(BH, S, D)
    v3 = value.reshape(BH, S, D)
    projp = jnp.pad(proj_dir, ((0, 0), (0, BLK - NPROJ)))

    rq, sq, sk = _rank_call(q3, k3, projp)

    sampled = jax.random.randint(jax.random.key(42), (B, H, SAMP), 0, S)
    samp_flat = (sampled.reshape(BH, SAMP)
                 + jnp.arange(BH, dtype=jnp.int32)[:, None] * S)
    sdiv = (sampled // BLK).reshape(BH, 1, SAMP)

    qp, kp, vp = _pad_call(q3, k3, v3)
    qs, ks, vs, ksub, vsub = _sc_sort_call(
        qp.reshape(BH * S, BLK), kp.reshape(BH * S, BLK),
        vp.reshape(BH * S, BLK), sq, sk, samp_flat)

    attn_s = _attn_call(qs.reshape(BH, S, BLK), ks.reshape(BH, S, BLK),
                        vs.reshape(BH, S, BLK), ksub.reshape(BH, SAMP, BLK),
                        vsub.reshape(BH, SAMP, BLK), sdiv, D)

    out = attn_s.reshape(BH * S, BLK)[rq.reshape(-1)]
    return out[:, :D].reshape(B, H, S, D)
